# baseline (speedup 1.0000x reference)
import dataclasses
import functools

import jax
import jax.numpy as jnp
from jax import lax
from jax.experimental import pallas as pl
from jax.experimental.pallas import tpu as pltpu
from jax.experimental.pallas import tpu_sc as plsc

R = 16384
C = 1000
B = 200
NC, NS, L = 2, 16, 16
NW = NC * NS
NCHUNK = 4
RCH = R // NCHUNK
ROWS_PER_W = RCH // NW
BLK = 32
NBLK = ROWS_PER_W // BLK
FULL = B // L
TAIL = B - L


def _make_chunk_kernel():
    mesh = plsc.VectorSubcoreMesh(core_axis_name="c", subcore_axis_name="s")
    cp = pltpu.CompilerParams()
    if "needs_layout_passes" in pltpu.CompilerParams.__dataclass_fields__:
        cp = dataclasses.replace(cp, needs_layout_passes=False)

    @functools.partial(
        pl.kernel,
        compiler_params=cp,
        out_type=jax.ShapeDtypeStruct((RCH, B), jnp.float32),
        mesh=mesh,
        scratch_types=[
            pltpu.VMEM((2, BLK, C), jnp.float32),
            pltpu.VMEM((2, BLK, B), jnp.int32),
            pltpu.VMEM((2, BLK, B), jnp.float32),
            pltpu.SemaphoreType.DMA((2,)),
            pltpu.SemaphoreType.DMA((2,)),
            pltpu.SemaphoreType.DMA((2,)),
        ],
    )
    def k(tbl_hbm, idx_hbm, out_hbm, rows_v, idx_v, out_v, st_, si_, so_):
        wid = lax.axis_index("s") * NC + lax.axis_index("c")

        def in_copies(g, b):
            blk0 = wid * ROWS_PER_W + g * BLK
            return (
                pltpu.make_async_copy(
                    tbl_hbm.at[pl.ds(blk0, BLK)], rows_v.at[b], st_.at[b]),
                pltpu.make_async_copy(
                    idx_hbm.at[pl.ds(blk0, BLK)], idx_v.at[b], si_.at[b]),
            )

        def out_copy(g, b):
            blk0 = wid * ROWS_PER_W + g * BLK
            return pltpu.make_async_copy(
                out_v.at[b], out_hbm.at[pl.ds(blk0, BLK)], so_.at[b])

        for c_ in in_copies(0, 0):
            c_.start()

        @pl.loop(0, NBLK)
        def _(g):
            b = lax.rem(g, 2)
            nb = 1 - b

            @pl.when(g >= 2)
            def _():
                out_copy(g - 2, b).wait()

            @pl.when(g + 1 < NBLK)
            def _():
                for c_ in in_copies(g + 1, nb):
                    c_.start()

            for c_ in in_copies(g, b):
                c_.wait()

            rows_b = rows_v.at[b]
            idx_b = idx_v.at[b]
            out_b = out_v.at[b]

            @pl.loop(0, BLK)
            def _(r):
                rsplat = jnp.full((L,), r, jnp.int32)
                offs = [c * L for c in range(FULL)] + [TAIL]
                for o in offs:
                    s = pl.ds(o, L)
                    col = idx_b[r, s]
                    out_b[r, s] = plsc.load_gather(rows_b, [rsplat, col])

            out_copy(g, b).start()

        out_copy(NBLK - 2, lax.rem(NBLK - 2, 2)).wait()
        out_copy(NBLK - 1, lax.rem(NBLK - 1, 2)).wait()

    return k


def kernel(input1, input2):
    idx = input2.astype(jnp.int32)
    k = _make_chunk_kernel()
    outs = []
    for ch in range(NCHUNK):
        lo = ch * RCH
        outs.append(k(input1[lo:lo + RCH], idx[lo:lo + RCH]))
    return jnp.concatenate(outs, axis=0)


# speedup vs baseline: 1.6955x; 1.6955x over previous
import dataclasses
import functools

import jax
import jax.numpy as jnp
from jax import lax
from jax.experimental import pallas as pl
from jax.experimental.pallas import tpu as pltpu
from jax.experimental.pallas import tpu_sc as plsc

R = 16384
C = 1000
B = 200
NC, NS, L = 2, 16, 16
NW = NC * NS
ROWS_PER_W = R // NW
BLK = 32
NBLK = ROWS_PER_W // BLK
FULL = B // L
TAIL = B - L
OFFS = [c * L for c in range(FULL)] + [TAIL]


def kernel(input1, input2):
    idx = input2.astype(jnp.int32)

    mesh = plsc.VectorSubcoreMesh(core_axis_name="c", subcore_axis_name="s")
    cp = pltpu.CompilerParams()
    fields = pltpu.CompilerParams.__dataclass_fields__
    if "needs_layout_passes" in fields:
        cp = dataclasses.replace(cp, needs_layout_passes=False)
    if "disable_bounds_checks" in fields:
        cp = dataclasses.replace(cp, disable_bounds_checks=True)

    @functools.partial(
        pl.kernel,
        compiler_params=cp,
        out_type=jax.ShapeDtypeStruct((R, B), jnp.float32),
        mesh=mesh,
        scratch_types=[
            pltpu.VMEM((2, BLK, C), jnp.float32),
            pltpu.VMEM((2, BLK, B), jnp.int32),
            pltpu.VMEM((2, BLK, B), jnp.float32),
            pltpu.SemaphoreType.DMA((2,)),
            pltpu.SemaphoreType.DMA((2,)),
            pltpu.SemaphoreType.DMA((2,)),
        ],
    )
    def k(tbl_hbm, idx_hbm, out_hbm, rows_v, idx_v, out_v, st_, si_, so_):
        wid = lax.axis_index("s") * NC + lax.axis_index("c")

        def in_copies(g, b):
            blk0 = wid * ROWS_PER_W + g * BLK
            return (
                pltpu.make_async_copy(
                    tbl_hbm.at[pl.ds(blk0, BLK)], rows_v.at[b], st_.at[b]),
                pltpu.make_async_copy(
                    idx_hbm.at[pl.ds(blk0, BLK)], idx_v.at[b], si_.at[b]),
            )

        def out_copy(g, b):
            blk0 = wid * ROWS_PER_W + g * BLK
            return pltpu.make_async_copy(
                out_v.at[b], out_hbm.at[pl.ds(blk0, BLK)], so_.at[b])

        for c_ in in_copies(0, 0):
            c_.start()

        @pl.loop(0, NBLK)
        def _(g):
            b = lax.rem(g, 2)
            nb = 1 - b

            @pl.when(g >= 2)
            def _():
                out_copy(g - 2, b).wait()

            @pl.when(g + 1 < NBLK)
            def _():
                for c_ in in_copies(g + 1, nb):
                    c_.start()

            for c_ in in_copies(g, b):
                c_.wait()

            rows_b = rows_v.at[b]
            idx_b = idx_v.at[b]
            out_b = out_v.at[b]

            @pl.loop(0, BLK, step=2)
            def _(r):
                work = []
                for dr in (0, 1):
                    rsplat = jnp.full((L,), r + dr, jnp.int32)
                    for o in OFFS:
                        work.append((dr, o, rsplat))
                cols = [(dr, o, rsplat, idx_b[r + dr, pl.ds(o, L)])
                        for (dr, o, rsplat) in work]
                vals = [(dr, o, plsc.load_gather(rows_b, [rsplat, col]))
                        for (dr, o, rsplat, col) in cols]
                for (dr, o, v) in vals:
                    out_b[r + dr, pl.ds(o, L)] = v

            out_copy(g, b).start()

        out_copy(NBLK - 2, lax.rem(NBLK - 2, 2)).wait()
        out_copy(NBLK - 1, lax.rem(NBLK - 1, 2)).wait()

    return k(input1, idx)


# speedup vs baseline: 2.0813x; 1.2276x over previous
import dataclasses
import functools

import jax
import jax.numpy as jnp
from jax import lax
from jax.experimental import pallas as pl
from jax.experimental.pallas import tpu as pltpu
from jax.experimental.pallas import tpu_sc as plsc

R = 16384
C = 1000
B = 200
NC, NS, L = 2, 16, 16
NW = NC * NS
ROWS_PER_W = R // NW
BLK = 32
NBLK = ROWS_PER_W // BLK
STR = 128
BPS = STR // BLK


def kernel(input1, input2):
    idx_t = input2.astype(jnp.int32).T
    mesh = plsc.VectorSubcoreMesh(core_axis_name="c", subcore_axis_name="s")
    cp = pltpu.CompilerParams()
    fields = pltpu.CompilerParams.__dataclass_fields__
    if "needs_layout_passes" in fields:
        cp = dataclasses.replace(cp, needs_layout_passes=False)
    if "disable_bounds_checks" in fields:
        cp = dataclasses.replace(cp, disable_bounds_checks=True)

    @functools.partial(
        pl.kernel,
        compiler_params=cp,
        out_type=jax.ShapeDtypeStruct((B, R), jnp.float32),
        mesh=mesh,
        scratch_types=[
            pltpu.VMEM((2, BLK, C), jnp.float32),
            pltpu.VMEM((B, STR), jnp.int32),
            pltpu.VMEM((B, STR), jnp.float32),
            pltpu.SemaphoreType.DMA((2,)),
        ],
    )
    def k(tbl_hbm, idx_hbm, out_hbm, rows_v, idx_v, out_v, st_):
        wid = lax.axis_index("s") * NC + lax.axis_index("c")
        row0 = wid * ROWS_PER_W

        rowvec = [lax.iota(jnp.int32, L) + ic * L for ic in range(BLK // L)]

        def tbl_copy(g, bslot):
            return pltpu.make_async_copy(
                tbl_hbm.at[pl.ds(row0 + g * BLK, BLK)],
                rows_v.at[bslot], st_.at[bslot])

        tbl_copy(0, 0).start()

        @pl.loop(0, NBLK)
        def _(g):
            b = lax.rem(g, 2)
            tbrel = lax.rem(g, BPS)
            stripe0 = row0 + (g // BPS) * STR

            @pl.when(tbrel == 0)
            def _():
                pltpu.sync_copy(idx_hbm.at[:, pl.ds(stripe0, STR)], idx_v)

            @pl.when(g + 1 < NBLK)
            def _():
                tbl_copy(g + 1, 1 - b).start()

            tbl_copy(g, b).wait()
            rows_b = rows_v.at[b]

            @pl.loop(0, B, step=2)
            def _(j):
                work = []
                for dj in (0, 1):
                    for ic in range(BLK // L):
                        o = tbrel * BLK + ic * L
                        work.append(
                            (dj, o, rowvec[ic], idx_v[j + dj, pl.ds(o, L)]))
                vals = [(dj, o, plsc.load_gather(rows_b, [rv, col]))
                        for (dj, o, rv, col) in work]
                for (dj, o, v) in vals:
                    out_v[j + dj, pl.ds(o, L)] = v

            @pl.when(tbrel == BPS - 1)
            def _():
                pltpu.sync_copy(out_v, out_hbm.at[:, pl.ds(stripe0, STR)])

    out_t = k(input1, idx_t)
    return out_t.T
